# Initial kernel scaffold; baseline (speedup 1.0000x reference)
import jax
import jax.numpy as jnp
from jax.experimental import pallas as pl


def kernel(x):
    raise NotImplementedError("write your pallas kernel here")


# speedup vs baseline: 3.2195x; 3.2195x over previous
import jax
import jax.numpy as jnp
from jax import lax
from jax.experimental import pallas as pl
from jax.experimental.pallas import tpu as pltpu

B, N, D = 4, 8192, 2048
W = 256
C = 256
NCHUNK = N // C


def _cumsum_rows(v):
    n, w = v.shape
    s = 1
    while s < n:
        v = v + jnp.concatenate(
            [jnp.zeros((s, w), v.dtype), v[: n - s]], axis=0)
        s *= 2
    return v


def _band_kernel(x_ref, rv_ref, o_ref, pi_ref):
    rv = rv_ref[0]

    def p1(k, acc):
        a = jnp.abs(x_ref[0, pl.ds(k * C, C), :])
        return acc + jnp.sum(a, axis=0, keepdims=True)

    s1 = lax.fori_loop(0, NCHUNK, p1, jnp.zeros((1, W), jnp.float32))

    def p2(k, acc):
        a = jnp.abs(x_ref[0, pl.ds(k * C, C), :])
        pi = jnp.exp((a / s1) * 2.0)
        pi_ref[pl.ds(k * C, C), :] = pi
        return acc + jnp.sum(pi, axis=0, keepdims=True)

    s2 = lax.fori_loop(0, NCHUNK, p2, jnp.zeros((1, W), jnp.float32))

    def p3(k, state):
        carry, gacc = state
        th = pi_ref[pl.ds(k * C, C), :] / s2
        incum = _cumsum_rows(th)
        cum = incum + carry
        ct = cum + 0.001
        prev_incum = jnp.concatenate(
            [jnp.zeros((1, W), jnp.float32), incum[: C - 1]], axis=0)
        prev_ct = (prev_incum + carry) + 0.001
        row0 = lax.broadcasted_iota(jnp.int32, (C, W), 0) == 0
        prev_ct = jnp.where(row0 & (k == 0), -1.0, prev_ct)
        crossing = (ct > rv) & (prev_ct <= rv)
        a = jnp.abs(x_ref[0, pl.ds(k * C, C), :])
        gacc = gacc + jnp.sum(jnp.where(crossing, a, 0.0), axis=0,
                              keepdims=True)
        return cum[C - 1:C, :], gacc

    _, g = lax.fori_loop(
        0, NCHUNK, p3,
        (jnp.zeros((1, W), jnp.float32), jnp.zeros((1, W), jnp.float32)))

    def p4(k, _):
        v = x_ref[0, pl.ds(k * C, C), :]
        o_ref[0, pl.ds(k * C, C), :] = jnp.where(jnp.abs(v) >= g, v, 0.0)
        return 0

    lax.fori_loop(0, NCHUNK, p4, 0)


def kernel(x):
    rv = jax.random.uniform(jax.random.key(42), (B, D), dtype=x.dtype)
    rv = rv.reshape(B, 1, D)
    return pl.pallas_call(
        _band_kernel,
        grid=(B, D // W),
        in_specs=[
            pl.BlockSpec((1, N, W), lambda b, j: (b, 0, j)),
            pl.BlockSpec((1, 1, W), lambda b, j: (b, 0, j)),
        ],
        out_specs=pl.BlockSpec((1, N, W), lambda b, j: (b, 0, j)),
        out_shape=jax.ShapeDtypeStruct((B, N, D), x.dtype),
        scratch_shapes=[pltpu.VMEM((N, W), jnp.float32)],
        compiler_params=pltpu.CompilerParams(
            dimension_semantics=("parallel", "parallel")),
    )(x, rv)
